# baseline (speedup 1.0000x reference)
import functools

import jax
import jax.numpy as jnp
from jax import lax
from jax.experimental import pallas as pl
from jax.experimental.pallas import tpu as pltpu
from jax.experimental.pallas import tpu_sc as plsc

NC = 2
NS = 16
L = 16
NW = NC * NS
B = 16384
D = 32
BPW = B // NW
NCHUNK = 4
CHUNK = BPW // NCHUNK


def _gathers(ut, it, uidx_v, pidx_v, nidx_v, u_v, p_v, n_v, sem, start):
    def dma(d, table, idx_ref, dst):
        for j in range(NCHUNK):
            cp = pltpu.make_async_copy(
                table.at[d].at[idx_ref.at[pl.ds(j * CHUNK, CHUNK)]],
                dst.at[d, pl.ds(j * CHUNK, CHUNK)],
                sem,
            )
            if start:
                cp.start()
            else:
                cp.wait()

    def body(d, carry):
        dma(d, ut, uidx_v, u_v)
        dma(d, it, pidx_v, p_v)
        dma(d, it, nidx_v, n_v)
        return carry

    lax.fori_loop(0, D, body, 0)


def _bpr_body(ut, it, uidx, pidx, nidx, pos_out, neg_out,
              uidx_v, pidx_v, nidx_v, u_v, p_v, n_v, psc, nsc, sem):
    wid = lax.axis_index("s") * NC + lax.axis_index("c")
    base = wid * BPW

    pltpu.sync_copy(uidx.at[pl.ds(base, BPW)], uidx_v)
    pltpu.sync_copy(pidx.at[pl.ds(base, BPW)], pidx_v)
    pltpu.sync_copy(nidx.at[pl.ds(base, BPW)], nidx_v)

    _gathers(ut, it, uidx_v, pidx_v, nidx_v, u_v, p_v, n_v, sem, start=True)
    _gathers(ut, it, uidx_v, pidx_v, nidx_v, u_v, p_v, n_v, sem, start=False)

    def blk_body(b, carry):
        r0 = b * L
        pacc = jnp.zeros((L,), jnp.float32)
        nacc = jnp.zeros((L,), jnp.float32)
        for d in range(D):
            u = u_v[d, pl.ds(r0, L)]
            p = p_v[d, pl.ds(r0, L)]
            n = n_v[d, pl.ds(r0, L)]
            pacc = pacc + u * p
            nacc = nacc + u * n
        psc[pl.ds(r0, L)] = pacc
        nsc[pl.ds(r0, L)] = nacc
        return carry

    lax.fori_loop(0, BPW // L, blk_body, 0)

    pltpu.sync_copy(psc, pos_out.at[pl.ds(base, BPW)])
    pltpu.sync_copy(nsc, neg_out.at[pl.ds(base, BPW)])


_bpr_call = functools.partial(
    pl.kernel,
    out_type=(
        jax.ShapeDtypeStruct((B,), jnp.float32),
        jax.ShapeDtypeStruct((B,), jnp.float32),
    ),
    mesh=plsc.VectorSubcoreMesh(core_axis_name="c", subcore_axis_name="s"),
    compiler_params=pltpu.CompilerParams(
        needs_layout_passes=False, use_tc_tiling_on_sc=False
    ),
    scratch_types=[
        pltpu.VMEM((BPW,), jnp.int32),
        pltpu.VMEM((BPW,), jnp.int32),
        pltpu.VMEM((BPW,), jnp.int32),
        pltpu.VMEM((D, BPW), jnp.float32),
        pltpu.VMEM((D, BPW), jnp.float32),
        pltpu.VMEM((D, BPW), jnp.float32),
        pltpu.VMEM((BPW,), jnp.float32),
        pltpu.VMEM((BPW,), jnp.float32),
        pltpu.SemaphoreType.DMA,
    ],
)(_bpr_body)


@jax.jit
def kernel(user_table, item_table, user_inputs, pos_inputs, neg_inputs):
    ut = user_table.T
    it = item_table.T
    uidx = user_inputs.reshape(B).astype(jnp.int32)
    pidx = pos_inputs.reshape(B).astype(jnp.int32)
    nidx = neg_inputs.reshape(B).astype(jnp.int32)
    pos, neg = _bpr_call(ut, it, uidx, pidx, nidx)
    return pos.reshape(B, 1), neg.reshape(B, 1)


# speedup vs baseline: 15.3240x; 15.3240x over previous
import functools

import jax
import jax.numpy as jnp
from jax import lax
from jax.experimental import pallas as pl
from jax.experimental.pallas import tpu as pltpu
from jax.experimental.pallas import tpu_sc as plsc

VOCAB = 1000000
NBLK = 7816
DSTRIDE = NBLK * 128
NC = 2
NS = 16
L = 16
NW = NC * NS
B = 16384
D = 32
BPW = B // NW
NCHUNK = 4
CHUNK = BPW // NCHUNK


def _gathers(ut, it, uidx_v, pidx_v, nidx_v, u_v, p_v, n_v, sem, start):
    def dma(d, table, idx_ref, dst):
        for j in range(NCHUNK):
            cp = pltpu.make_async_copy(
                table.at[pl.ds(d * DSTRIDE, DSTRIDE)].at[
                    idx_ref.at[pl.ds(j * CHUNK, CHUNK)]
                ],
                dst.at[d, pl.ds(j * CHUNK, CHUNK)],
                sem,
            )
            if start:
                cp.start()
            else:
                cp.wait()

    def body(d, carry):
        dma(d, ut, uidx_v, u_v)
        dma(d, it, pidx_v, p_v)
        dma(d, it, nidx_v, n_v)
        return carry

    lax.fori_loop(0, D, body, 0)


def _bpr_body(ut, it, uidx, pidx, nidx, pos_out, neg_out,
              uidx_v, pidx_v, nidx_v, u_v, p_v, n_v, psc, nsc, sem):
    wid = lax.axis_index("s") * NC + lax.axis_index("c")
    base = wid * BPW

    pltpu.sync_copy(uidx.at[pl.ds(base, BPW)], uidx_v)
    pltpu.sync_copy(pidx.at[pl.ds(base, BPW)], pidx_v)
    pltpu.sync_copy(nidx.at[pl.ds(base, BPW)], nidx_v)

    _gathers(ut, it, uidx_v, pidx_v, nidx_v, u_v, p_v, n_v, sem, start=True)
    _gathers(ut, it, uidx_v, pidx_v, nidx_v, u_v, p_v, n_v, sem, start=False)

    def blk_body(b, carry):
        r0 = b * L
        pacc = jnp.zeros((L,), jnp.float32)
        nacc = jnp.zeros((L,), jnp.float32)
        for d in range(D):
            u = u_v[d, pl.ds(r0, L)]
            p = p_v[d, pl.ds(r0, L)]
            n = n_v[d, pl.ds(r0, L)]
            pacc = pacc + u * p
            nacc = nacc + u * n
        psc[pl.ds(r0, L)] = pacc
        nsc[pl.ds(r0, L)] = nacc
        return carry

    lax.fori_loop(0, BPW // L, blk_body, 0)

    pltpu.sync_copy(psc, pos_out.at[pl.ds(base, BPW)])
    pltpu.sync_copy(nsc, neg_out.at[pl.ds(base, BPW)])


_bpr_call = functools.partial(
    pl.kernel,
    out_type=(
        jax.ShapeDtypeStruct((B,), jnp.float32),
        jax.ShapeDtypeStruct((B,), jnp.float32),
    ),
    mesh=plsc.VectorSubcoreMesh(core_axis_name="c", subcore_axis_name="s"),
    compiler_params=pltpu.CompilerParams(
        needs_layout_passes=False, use_tc_tiling_on_sc=False
    ),
    scratch_types=[
        pltpu.VMEM((BPW,), jnp.int32),
        pltpu.VMEM((BPW,), jnp.int32),
        pltpu.VMEM((BPW,), jnp.int32),
        pltpu.VMEM((D, BPW), jnp.float32),
        pltpu.VMEM((D, BPW), jnp.float32),
        pltpu.VMEM((D, BPW), jnp.float32),
        pltpu.VMEM((BPW,), jnp.float32),
        pltpu.VMEM((BPW,), jnp.float32),
        pltpu.SemaphoreType.DMA,
    ],
)(_bpr_body)


TCW = 8192
JW = TCW // 128
TGRID = -(-VOCAB // TCW)


def _untile_body(in_ref, out_ref):
    out_ref[...] = in_ref[...].reshape(D, JW, 128)


_untile = pl.pallas_call(
    _untile_body,
    grid=(TGRID,),
    in_specs=[pl.BlockSpec((D, TCW), lambda k: (0, k))],
    out_specs=pl.BlockSpec((D, JW, 128), lambda k: (0, k, 0)),
    out_shape=jax.ShapeDtypeStruct((D, NBLK, 128), jnp.float32),
)


@jax.jit
def kernel(user_table, item_table, user_inputs, pos_inputs, neg_inputs):
    ut = _untile(user_table.T).reshape(D * DSTRIDE)
    it = _untile(item_table.T).reshape(D * DSTRIDE)
    uidx = user_inputs.reshape(B).astype(jnp.int32)
    pidx = pos_inputs.reshape(B).astype(jnp.int32)
    nidx = neg_inputs.reshape(B).astype(jnp.int32)
    pos, neg = _bpr_call(ut, it, uidx, pidx, nidx)
    return pos.reshape(B, 1), neg.reshape(B, 1)
